# baseline (speedup 1.0000x reference)
import functools

import jax
import jax.numpy as jnp
from jax import lax
from jax.experimental import pallas as pl
from jax.experimental.pallas import tpu as pltpu
from jax.experimental.pallas import tpu_sc as plsc

B, C, H, W = 4, 19, 512, 512
N = B * H * W
SHIFT = 18
NBIN = 4096
HR, HC = 64, 128
NCORES, NSUB = 2, 16
NTILES = NCORES * NSUB
PIX = N // NTILES
BH = 256


def _keys_body(x_ref, lab_ref, out_ref):
    x = x_ref[...]
    lab = lab_ref[...]
    m = jnp.max(x, axis=1, keepdims=True)
    ex = jnp.exp(x - m)
    p = ex / jnp.sum(ex, axis=1, keepdims=True)
    cls = lax.broadcasted_iota(jnp.int32, x.shape, 1)
    fg = lab == cls
    e = jnp.where(fg, 1.0 - p, p)
    bits = lax.bitcast_convert_type(e, jnp.int32)
    key = lax.shift_right_logical(bits, SHIFT) + jnp.where(fg, NBIN, 0)
    out_ref[...] = key.reshape(C, 1, BH, W)


def _compute_keys(x_src, labels4):
    return pl.pallas_call(
        _keys_body,
        grid=(B, H // BH),
        in_specs=[
            pl.BlockSpec((1, C, BH, W), lambda b, h: (b, 0, h, 0)),
            pl.BlockSpec((1, 1, BH, W), lambda b, h: (b, 0, h, 0)),
        ],
        out_specs=pl.BlockSpec((C, 1, BH, W), lambda b, h: (0, b, h, 0)),
        out_shape=jax.ShapeDtypeStruct((C, B, H, W), jnp.int32),
    )(x_src, labels4)


NHIST = 2 * NBIN


def _sc_hist_body(keys_hbm, zeros_hbm, out_hbm, kbuf0, kbuf1, h0, h1, h2,
                  ks0, ks1, os0, os1, os2, zs0, zs1, zs2):
    core = lax.axis_index("c")
    sub = lax.axis_index("s")
    wid = core * NSUB + sub
    bb = lax.shift_right_logical(wid, 3)
    r0 = jnp.bitwise_and(wid, 7) * (PIX // W)
    kbufs = (kbuf0, kbuf1)
    hists = (h0, h1, h2)
    ksems = (ks0, ks1)
    osems = (os0, os1, os2)
    zsems = (zs0, zs1, zs2)

    ones16 = jnp.ones((16,), jnp.float32)

    def keys_copy(cls, kb):
        return pltpu.async_copy(
            keys_hbm.at[cls, bb, pl.ds(r0, PIX // W), :], kbufs[kb], ksems[kb])

    def zero_copy(hb):
        return pltpu.async_copy(zeros_hbm, hists[hb], zsems[hb])

    kpend = [keys_copy(0, 0), None]
    zpend = [zero_copy(0), zero_copy(1), zero_copy(2)]
    opend = [None, None, None]
    for cls in range(C):
        kb = cls & 1
        hb = cls % 3
        kbuf = kbufs[kb]
        hist = hists[hb]
        kpend[kb].wait()
        if cls + 1 < C:
            kpend[1 - kb] = keys_copy(cls + 1, 1 - kb)
        zpend[hb].wait()

        @plsc.parallel_loop(0, PIX // 16, 1, unroll=16)
        def _(i):
            k = kbuf[lax.shift_right_logical(i, 5),
                     pl.ds(jnp.bitwise_and(i, 31) * 16, 16)]
            plsc.addupdate_scatter(hist, [k], ones16)

        opend[hb] = pltpu.async_copy(
            hist, out_hbm.at[pl.ds((wid * C + cls) * NHIST, NHIST)],
            osems[hb])
        pb = (cls - 1) % 3
        if cls >= 1 and opend[pb] is not None and cls + 2 < C:
            opend[pb].wait()
            opend[pb] = None
            zpend[pb] = zero_copy(pb)
    for hb in range(3):
        if opend[hb] is not None:
            opend[hb].wait()


@functools.cache
def _sc_hist_kernel():
    return pl.kernel(
        _sc_hist_body,
        out_type=jax.ShapeDtypeStruct((NTILES * C * NHIST,), jnp.float32),
        name="sc_hist",
        mesh=plsc.VectorSubcoreMesh(core_axis_name="c", subcore_axis_name="s",
                                    num_cores=NCORES, num_subcores=NSUB),
        compiler_params=pltpu.CompilerParams(needs_layout_passes=False),
        scratch_types=[
            pltpu.VMEM((PIX // W, W), jnp.int32),
            pltpu.VMEM((PIX // W, W), jnp.int32),
            pltpu.VMEM((NHIST,), jnp.float32),
            pltpu.VMEM((NHIST,), jnp.float32),
            pltpu.VMEM((NHIST,), jnp.float32),
            pltpu.SemaphoreType.DMA,
            pltpu.SemaphoreType.DMA,
            pltpu.SemaphoreType.DMA,
            pltpu.SemaphoreType.DMA,
            pltpu.SemaphoreType.DMA,
            pltpu.SemaphoreType.DMA,
            pltpu.SemaphoreType.DMA,
            pltpu.SemaphoreType.DMA,
        ],
    )


def _loss_body(h_ref, out_ref):
    nrows = HR // 2
    li = lax.broadcasted_iota(jnp.int32, (HC, HC), 0)
    lj = lax.broadcasted_iota(jnp.int32, (HC, HC), 1)
    m_suffix = (li >= lj).astype(jnp.float32)
    qi = lax.broadcasted_iota(jnp.int32, (nrows, nrows), 0)
    qj = lax.broadcasted_iota(jnp.int32, (nrows, nrows), 1)
    q_after = (qj > qi).astype(jnp.float32)

    r_iota = lax.broadcasted_iota(jnp.int32, (nrows, HC), 0)
    l_iota = lax.broadcasted_iota(jnp.int32, (nrows, HC), 1)
    bins = jnp.minimum(r_iota * HC + l_iota, 0x3F800000 >> SHIFT)
    cbits = (bins << SHIFT) | (1 << (SHIFT - 1))
    centers = jnp.minimum(lax.bitcast_convert_type(cbits, jnp.float32), 1.0)

    def suffix(x):
        inrow = jnp.dot(x, m_suffix, preferred_element_type=jnp.float32)
        rowtot = inrow[:, 0:1]
        after = jnp.dot(q_after, rowtot, preferred_element_type=jnp.float32)
        return inrow + after

    def tbody(t, acc):
        return acc + h_ref[t, 0]
    a = lax.fori_loop(0, NTILES, tbody, jnp.zeros((HR, HC), jnp.float32))

    n1 = a[nrows:]
    n = a[:nrows] + n1
    g = n1
    sn = suffix(n)
    sg = suffix(g)
    gtot = jnp.sum(g)

    def jac(cn, cg):
        u = jnp.maximum(gtot + cn - cg, 1.0)
        return jnp.where(cn > 0, 1.0 - (gtot - cg) / u, 0.0)

    delta = jac(sn, sg) - jac(sn - n, sg - g)
    loss_c = jnp.sum(centers * delta)

    @pl.when(pl.program_id(0) == 0)
    def _():
        out_ref[0, 0] = 0.0
    out_ref[0, 0] += loss_c / C


def _lovasz_from_hist(hist):
    return pl.pallas_call(
        _loss_body,
        grid=(C,),
        in_specs=[pl.BlockSpec((NTILES, 1, HR, HC), lambda c: (0, c, 0, 0))],
        out_specs=pl.BlockSpec(memory_space=pltpu.SMEM),
        out_shape=jax.ShapeDtypeStruct((1, 1), jnp.float32),
    )(hist)


def kernel(x_src, x_tgt):
    labels4 = x_tgt.reshape(B, 1, H, W)
    keys = _compute_keys(x_src, labels4)
    zeros = jnp.zeros((NHIST,), jnp.float32)
    hist = _sc_hist_kernel()(keys, zeros).reshape(NTILES, C, HR, HC)
    return _lovasz_from_hist(hist).reshape(())


# speedup vs baseline: 1.0022x; 1.0022x over previous
import functools

import jax
import jax.numpy as jnp
from jax import lax
from jax.experimental import pallas as pl
from jax.experimental.pallas import tpu as pltpu
from jax.experimental.pallas import tpu_sc as plsc

B, C, H, W = 4, 19, 512, 512
N = B * H * W
SHIFT = 18
NBIN = 4096
HR, HC = 64, 128
NCORES, NSUB = 2, 16
NTILES = NCORES * NSUB
PIX = N // NTILES
BH = 256


def _keys_body(x_ref, lab_ref, out_ref):
    x = x_ref[...]
    lab = lab_ref[...]
    m = jnp.max(x, axis=1, keepdims=True)
    ex = jnp.exp(x - m)
    p = ex / jnp.sum(ex, axis=1, keepdims=True)
    cls = lax.broadcasted_iota(jnp.int32, x.shape, 1)
    fg = lab == cls
    e = jnp.where(fg, 1.0 - p, p)
    bits = lax.bitcast_convert_type(e, jnp.int32)
    key = lax.shift_right_logical(bits, SHIFT) + jnp.where(fg, NBIN, 0)
    out_ref[...] = key.reshape(C, 1, BH, W)


def _compute_keys(x_src, labels4):
    return pl.pallas_call(
        _keys_body,
        grid=(B, H // BH),
        in_specs=[
            pl.BlockSpec((1, C, BH, W), lambda b, h: (b, 0, h, 0)),
            pl.BlockSpec((1, 1, BH, W), lambda b, h: (b, 0, h, 0)),
        ],
        out_specs=pl.BlockSpec((C, 1, BH, W), lambda b, h: (0, b, h, 0)),
        out_shape=jax.ShapeDtypeStruct((C, B, H, W), jnp.int32),
    )(x_src, labels4)


NHIST = 2 * NBIN


def _sc_hist_body(keys_hbm, zeros_hbm, out_hbm, kbuf0, kbuf1, h0, h1, h2,
                  ks0, ks1, os0, os1, os2, zs0, zs1, zs2):
    core = lax.axis_index("c")
    sub = lax.axis_index("s")
    wid = core * NSUB + sub
    bb = lax.shift_right_logical(wid, 3)
    r0 = jnp.bitwise_and(wid, 7) * (PIX // W)
    kbufs = (kbuf0, kbuf1)
    hists = (h0, h1, h2)
    ksems = (ks0, ks1)
    osems = (os0, os1, os2)
    zsems = (zs0, zs1, zs2)

    ones16 = jnp.ones((16,), jnp.float32)

    def keys_copy(cls, kb):
        return pltpu.async_copy(
            keys_hbm.at[cls, bb, pl.ds(r0, PIX // W), :], kbufs[kb], ksems[kb])

    def zero_copy(hb):
        return pltpu.async_copy(zeros_hbm, hists[hb], zsems[hb])

    kpend = [keys_copy(0, 0), None]
    zpend = [zero_copy(0), zero_copy(1), zero_copy(2)]
    opend = [None, None, None]
    for cls in range(C):
        kb = cls & 1
        hb = cls % 3
        kbuf = kbufs[kb]
        hist = hists[hb]
        kpend[kb].wait()
        if cls + 1 < C:
            kpend[1 - kb] = keys_copy(cls + 1, 1 - kb)
        zpend[hb].wait()

        @plsc.parallel_loop(0, PIX // 16, 1, unroll=8)
        def _(i):
            k = kbuf[lax.shift_right_logical(i, 5),
                     pl.ds(jnp.bitwise_and(i, 31) * 16, 16)]
            plsc.addupdate_scatter(hist, [k], ones16)

        opend[hb] = pltpu.async_copy(
            hist, out_hbm.at[pl.ds((wid * C + cls) * NHIST, NHIST)],
            osems[hb])
        pb = (cls - 1) % 3
        if cls >= 1 and opend[pb] is not None and cls + 2 < C:
            opend[pb].wait()
            opend[pb] = None
            zpend[pb] = zero_copy(pb)
    for hb in range(3):
        if opend[hb] is not None:
            opend[hb].wait()


@functools.cache
def _sc_hist_kernel():
    return pl.kernel(
        _sc_hist_body,
        out_type=jax.ShapeDtypeStruct((NTILES * C * NHIST,), jnp.float32),
        name="sc_hist",
        mesh=plsc.VectorSubcoreMesh(core_axis_name="c", subcore_axis_name="s",
                                    num_cores=NCORES, num_subcores=NSUB),
        compiler_params=pltpu.CompilerParams(needs_layout_passes=False),
        scratch_types=[
            pltpu.VMEM((PIX // W, W), jnp.int32),
            pltpu.VMEM((PIX // W, W), jnp.int32),
            pltpu.VMEM((NHIST,), jnp.float32),
            pltpu.VMEM((NHIST,), jnp.float32),
            pltpu.VMEM((NHIST,), jnp.float32),
            pltpu.SemaphoreType.DMA,
            pltpu.SemaphoreType.DMA,
            pltpu.SemaphoreType.DMA,
            pltpu.SemaphoreType.DMA,
            pltpu.SemaphoreType.DMA,
            pltpu.SemaphoreType.DMA,
            pltpu.SemaphoreType.DMA,
            pltpu.SemaphoreType.DMA,
        ],
    )


def _loss_body(h_ref, out_ref):
    nrows = HR // 2
    li = lax.broadcasted_iota(jnp.int32, (HC, HC), 0)
    lj = lax.broadcasted_iota(jnp.int32, (HC, HC), 1)
    m_suffix = (li >= lj).astype(jnp.float32)
    qi = lax.broadcasted_iota(jnp.int32, (nrows, nrows), 0)
    qj = lax.broadcasted_iota(jnp.int32, (nrows, nrows), 1)
    q_after = (qj > qi).astype(jnp.float32)

    r_iota = lax.broadcasted_iota(jnp.int32, (nrows, HC), 0)
    l_iota = lax.broadcasted_iota(jnp.int32, (nrows, HC), 1)
    bins = jnp.minimum(r_iota * HC + l_iota, 0x3F800000 >> SHIFT)
    cbits = (bins << SHIFT) | (1 << (SHIFT - 1))
    centers = jnp.minimum(lax.bitcast_convert_type(cbits, jnp.float32), 1.0)

    def suffix(x):
        inrow = jnp.dot(x, m_suffix, preferred_element_type=jnp.float32)
        rowtot = inrow[:, 0:1]
        after = jnp.dot(q_after, rowtot, preferred_element_type=jnp.float32)
        return inrow + after

    def tbody(t, acc):
        return acc + h_ref[t, 0]
    a = lax.fori_loop(0, NTILES, tbody, jnp.zeros((HR, HC), jnp.float32))

    n1 = a[nrows:]
    n = a[:nrows] + n1
    g = n1
    sn = suffix(n)
    sg = suffix(g)
    gtot = jnp.sum(g)

    def jac(cn, cg):
        u = jnp.maximum(gtot + cn - cg, 1.0)
        return jnp.where(cn > 0, 1.0 - (gtot - cg) / u, 0.0)

    delta = jac(sn, sg) - jac(sn - n, sg - g)
    loss_c = jnp.sum(centers * delta)

    @pl.when(pl.program_id(0) == 0)
    def _():
        out_ref[0, 0] = 0.0
    out_ref[0, 0] += loss_c / C


def _lovasz_from_hist(hist):
    return pl.pallas_call(
        _loss_body,
        grid=(C,),
        in_specs=[pl.BlockSpec((NTILES, 1, HR, HC), lambda c: (0, c, 0, 0))],
        out_specs=pl.BlockSpec(memory_space=pltpu.SMEM),
        out_shape=jax.ShapeDtypeStruct((1, 1), jnp.float32),
    )(hist)


def kernel(x_src, x_tgt):
    labels4 = x_tgt.reshape(B, 1, H, W)
    keys = _compute_keys(x_src, labels4)
    zeros = jnp.zeros((NHIST,), jnp.float32)
    hist = _sc_hist_kernel()(keys, zeros).reshape(NTILES, C, HR, HC)
    return _lovasz_from_hist(hist).reshape(())
